# baseline (speedup 1.0000x reference)
import functools

import jax
import jax.numpy as jnp
from jax import lax
from jax.experimental import pallas as pl
from jax.experimental.pallas import tpu as pltpu
from jax.experimental.pallas import tpu_sc as plsc

NC = 2
NS = 16
CHUNK = 128


@functools.cache
def _mesh():
    return plsc.VectorSubcoreMesh(core_axis_name="c", subcore_axis_name="s",
                                  num_cores=NC, num_subcores=NS)


def _ceil_to(x, m):
    return (x + m - 1) // m * m


def _sc_degree(dst2d_hbm, init_hbm, ones_hbm, out_hbm,
               acc_sp, dst_slab, ones_v, sem):
    c = lax.axis_index("c")
    s = lax.axis_index("s")
    np_rows = acc_sp.shape[0]
    rows_per = np_rows // NS
    n_chunks = dst2d_hbm.shape[0] // (NC * NS)
    r0 = s * rows_per
    w = c * NS + s
    pltpu.sync_copy(dst2d_hbm.at[pl.ds(w * n_chunks, n_chunks)], dst_slab)
    pltpu.sync_copy(init_hbm.at[c, pl.ds(r0, rows_per)],
                    acc_sp.at[pl.ds(r0, rows_per)])
    pltpu.sync_copy(ones_hbm, ones_v)
    plsc.subcore_barrier()

    grp = 8

    def body(j, carry):
        for b in range(grp):
            pltpu.async_copy(ones_v, acc_sp.at[dst_slab.at[j * grp + b]],
                             sem, add=True)
        for b in range(grp):
            pltpu.make_async_copy(ones_v, acc_sp.at[dst_slab.at[0]],
                                  sem).wait()
        return carry

    lax.fori_loop(0, n_chunks // grp, body, 0)
    plsc.subcore_barrier()
    pltpu.sync_copy(acc_sp.at[pl.ds(r0, rows_per)],
                    out_hbm.at[c, pl.ds(r0, rows_per)])


def _edge_pipeline(table_ref, acc_sp, src_slab, dst_slab, bufs,
                   gsems, n_chunks):
    def gather_desc(k, b):
        return pltpu.make_async_copy(table_ref.at[src_slab.at[k]],
                                     bufs.at[b], gsems[b])

    gather_desc(0, 0).start()

    def body(j, carry):
        for b in range(2):
            k = j * 2 + b
            nb = 1 - b

            @pl.when(k + 1 < n_chunks)
            def _():
                gather_desc(k + 1, nb).start()

            gather_desc(k, b).wait()
            pltpu.sync_copy(bufs.at[b], acc_sp.at[dst_slab.at[k]], add=True)
        return carry

    lax.fori_loop(0, n_chunks // 2, body, 0)


def _sc_edges_wide(table_hbm, src2d_hbm, dst2d_hbm, out_hbm,
                   acc_sp, src_slab, dst_slab, bufs, gs0, gs1):
    c = lax.axis_index("c")
    s = lax.axis_index("s")
    np_rows = acc_sp.shape[0]
    rows_per = np_rows // NS
    n_chunks = src2d_hbm.shape[0] // NS
    r0 = s * rows_per
    pltpu.sync_copy(src2d_hbm.at[pl.ds(s * n_chunks, n_chunks)], src_slab)
    pltpu.sync_copy(dst2d_hbm.at[pl.ds(s * n_chunks, n_chunks)], dst_slab)
    pltpu.sync_copy(table_hbm.at[c].at[pl.ds(r0, rows_per)],
                    acc_sp.at[pl.ds(r0, rows_per)])
    plsc.subcore_barrier()
    _edge_pipeline(table_hbm.at[c], acc_sp, src_slab, dst_slab, bufs,
                   [gs0, gs1], n_chunks)
    plsc.subcore_barrier()
    pltpu.sync_copy(acc_sp.at[pl.ds(r0, rows_per)],
                    out_hbm.at[c, pl.ds(r0, rows_per)])


def _sc_edges_narrow(z_hbm, src2d_hbm, dst2d_hbm, out_hbm,
                     acc_sp, src_slab, dst_slab, bufs, gs0, gs1):
    c = lax.axis_index("c")
    s = lax.axis_index("s")
    np_rows = acc_sp.shape[0]
    rows_per = np_rows // NS
    n_chunks = src2d_hbm.shape[0] // (NC * NS)
    r0 = s * rows_per
    w = c * NS + s
    pltpu.sync_copy(src2d_hbm.at[pl.ds(w * n_chunks, n_chunks)], src_slab)
    pltpu.sync_copy(dst2d_hbm.at[pl.ds(w * n_chunks, n_chunks)], dst_slab)
    pltpu.sync_copy(z_hbm.at[pl.ds(r0, rows_per)],
                    acc_sp.at[pl.ds(r0, rows_per)])
    plsc.subcore_barrier()
    _edge_pipeline(z_hbm, acc_sp, src_slab, dst_slab, bufs,
                   [gs0, gs1], n_chunks)
    plsc.subcore_barrier()
    pltpu.sync_copy(acc_sp.at[pl.ds(r0, rows_per)],
                    out_hbm.at[c, pl.ds(r0, rows_per)])


def _tc_scale_in(x_ref, w1_ref, deg_ref, t_ref, dis_ref):
    deg = deg_ref[0, :, 0:1] + deg_ref[1, :, 0:1]
    dis = lax.rsqrt(deg)
    h = jnp.dot(x_ref[...], w1_ref[...], precision=lax.Precision.HIGHEST,
                preferred_element_type=jnp.float32)
    hp = h * dis
    t_ref[0] = hp[:, :64]
    t_ref[1] = hp[:, 64:]
    dis_ref[...] = jnp.broadcast_to(dis, dis_ref.shape)


def _tc_mid(agg_ref, dis_ref, b1_ref, w2_ref, z_ref):
    dis = dis_ref[:, 0:1]
    agg = jnp.concatenate([agg_ref[0], agg_ref[1]], axis=1)
    out1 = jnp.maximum(agg * dis + b1_ref[...], 0.0)
    z_ref[...] = jnp.dot(out1 * dis, w2_ref[...],
                         precision=lax.Precision.HIGHEST,
                         preferred_element_type=jnp.float32)


def _tc_combine_pool(p_ref, z_ref, dis_ref, b2_ref, batch_ref,
                     out_ref, *, n, g):
    dis = dis_ref[:, 0:1]
    agg = p_ref[0] + p_ref[1] - z_ref[...]
    h2 = agg * dis + b2_ref[...]
    h2 = h2[:n, :]
    gids = lax.broadcasted_iota(jnp.int32, (g, n), 0)
    onehot = (gids == batch_ref[...]).astype(jnp.float32)
    out_ref[...] = jnp.dot(onehot, h2, precision=lax.Precision.HIGHEST,
                           preferred_element_type=jnp.float32)


def kernel(x, edge_index, batch, W1, b1, W2, b2):
    n, d_in = x.shape
    d_hid = W1.shape[1]
    d_out = W2.shape[1]
    g = 64
    e = edge_index.shape[1]
    f32 = jnp.float32

    n_pad = _ceil_to(n, NS * CHUNK)
    e_pad = _ceil_to(e, 2 * NC * NS * CHUNK)
    dh = d_hid // NC

    x_p = jnp.pad(x, ((0, n_pad - n), (0, 0)))
    src2d = jnp.pad(edge_index[0].astype(jnp.int32),
                    (0, e_pad - e)).reshape(e_pad // CHUNK, CHUNK)
    dst2d = jnp.pad(edge_index[1].astype(jnp.int32), (0, e_pad - e),
                    constant_values=n).reshape(e_pad // CHUNK, CHUNK)
    deg_init = jnp.stack([jnp.ones((n_pad, 16), f32),
                          jnp.zeros((n_pad, 16), f32)])
    ones_rows = jnp.ones((CHUNK, 16), f32)
    b1_r = b1.reshape(1, d_hid)
    b2_r = b2.reshape(1, d_out)
    batch_r = batch.astype(jnp.int32).reshape(1, n)

    n_chunks3 = e_pad // CHUNK // NS
    n_chunks5 = e_pad // CHUNK // (NC * NS)

    sc_params = pltpu.CompilerParams(use_tc_tiling_on_sc=False)

    deg_parts = pl.kernel(
        _sc_degree,
        out_type=jax.ShapeDtypeStruct((NC, n_pad, 16), f32),
        mesh=_mesh(),
        compiler_params=sc_params,
        scratch_types=[
            pltpu.VMEM_SHARED((n_pad, 16), f32),
            pltpu.VMEM((n_chunks5, CHUNK), jnp.int32),
            pltpu.VMEM((CHUNK, 16), f32),
            pltpu.SemaphoreType.DMA,
        ],
    )(dst2d, deg_init, ones_rows)

    rows_blk = n_pad // 8
    table, dis8 = pl.pallas_call(
        _tc_scale_in,
        grid=(8,),
        in_specs=[
            pl.BlockSpec((rows_blk, d_in), lambda i: (i, 0)),
            pl.BlockSpec((d_in, d_hid), lambda i: (0, 0)),
            pl.BlockSpec((NC, rows_blk, 16), lambda i: (0, i, 0)),
        ],
        out_specs=[
            pl.BlockSpec((NC, rows_blk, dh), lambda i: (0, i, 0)),
            pl.BlockSpec((rows_blk, 8), lambda i: (i, 0)),
        ],
        out_shape=[
            jax.ShapeDtypeStruct((NC, n_pad, dh), f32),
            jax.ShapeDtypeStruct((n_pad, 8), f32),
        ],
    )(x_p, W1, deg_parts)

    agg1 = pl.kernel(
        _sc_edges_wide,
        out_type=jax.ShapeDtypeStruct((NC, n_pad, dh), f32),
        mesh=_mesh(),
        compiler_params=sc_params,
        scratch_types=[
            pltpu.VMEM_SHARED((n_pad, dh), f32),
            pltpu.VMEM((n_chunks3, CHUNK), jnp.int32),
            pltpu.VMEM((n_chunks3, CHUNK), jnp.int32),
            pltpu.VMEM((2, CHUNK, dh), f32),
            pltpu.SemaphoreType.DMA,
            pltpu.SemaphoreType.DMA,
        ],
    )(table, src2d, dst2d)

    z = pl.pallas_call(
        _tc_mid,
        grid=(8,),
        in_specs=[
            pl.BlockSpec((NC, rows_blk, dh), lambda i: (0, i, 0)),
            pl.BlockSpec((rows_blk, 8), lambda i: (i, 0)),
            pl.BlockSpec((1, d_hid), lambda i: (0, 0)),
            pl.BlockSpec((d_hid, d_out), lambda i: (0, 0)),
        ],
        out_specs=pl.BlockSpec((rows_blk, d_out), lambda i: (i, 0)),
        out_shape=jax.ShapeDtypeStruct((n_pad, d_out), f32),
    )(agg1, dis8, b1_r, W2)

    agg2 = pl.kernel(
        _sc_edges_narrow,
        out_type=jax.ShapeDtypeStruct((NC, n_pad, d_out), f32),
        mesh=_mesh(),
        compiler_params=sc_params,
        scratch_types=[
            pltpu.VMEM_SHARED((n_pad, d_out), f32),
            pltpu.VMEM((n_chunks5, CHUNK), jnp.int32),
            pltpu.VMEM((n_chunks5, CHUNK), jnp.int32),
            pltpu.VMEM((2, CHUNK, d_out), f32),
            pltpu.SemaphoreType.DMA,
            pltpu.SemaphoreType.DMA,
        ],
    )(z, src2d, dst2d)

    out = pl.pallas_call(
        functools.partial(_tc_combine_pool, n=n, g=g),
        in_specs=[
            pl.BlockSpec((NC, n_pad, d_out), lambda: (0, 0, 0)),
            pl.BlockSpec((n_pad, d_out), lambda: (0, 0)),
            pl.BlockSpec((n_pad, 8), lambda: (0, 0)),
            pl.BlockSpec((1, d_out), lambda: (0, 0)),
            pl.BlockSpec((1, n), lambda: (0, 0)),
        ],
        out_specs=pl.BlockSpec((g, d_out), lambda: (0, 0)),
        out_shape=jax.ShapeDtypeStruct((g, d_out), f32),
    )(agg2, z, dis8, b2_r, batch_r)

    return out


# speedup vs baseline: 1.3198x; 1.3198x over previous
import functools

import jax
import jax.numpy as jnp
from jax import lax
from jax.experimental import pallas as pl
from jax.experimental.pallas import tpu as pltpu
from jax.experimental.pallas import tpu_sc as plsc

NC = 2
NS = 16
CHUNK = 128


@functools.cache
def _mesh():
    return plsc.VectorSubcoreMesh(core_axis_name="c", subcore_axis_name="s",
                                  num_cores=NC, num_subcores=NS)


def _ceil_to(x, m):
    return (x + m - 1) // m * m


def _sc_degree(dst2d_hbm, init_hbm, ones_hbm, out_hbm,
               acc_sp, dst_slab, ones_v, sem):
    c = lax.axis_index("c")
    s = lax.axis_index("s")
    np_rows = acc_sp.shape[0]
    rows_per = np_rows // NS
    n_chunks = dst2d_hbm.shape[0] // (NC * NS)
    r0 = s * rows_per
    w = c * NS + s
    pltpu.sync_copy(dst2d_hbm.at[pl.ds(w * n_chunks, n_chunks)], dst_slab)
    pltpu.sync_copy(init_hbm.at[c, pl.ds(r0, rows_per)],
                    acc_sp.at[pl.ds(r0, rows_per)])
    pltpu.sync_copy(ones_hbm, ones_v)
    plsc.subcore_barrier()

    grp = 8

    def body(j, carry):
        for b in range(grp):
            pltpu.async_copy(ones_v, acc_sp.at[dst_slab.at[j * grp + b]],
                             sem, add=True)
        for b in range(grp):
            pltpu.make_async_copy(ones_v, acc_sp.at[dst_slab.at[0]],
                                  sem).wait()
        return carry

    lax.fori_loop(0, n_chunks // grp, body, 0)
    plsc.subcore_barrier()
    pltpu.sync_copy(acc_sp.at[pl.ds(r0, rows_per)],
                    out_hbm.at[c, pl.ds(r0, rows_per)])


def _edge_pipeline(table_ref, acc_sp, src_slab, dst_slab, bufs,
                   gsems, n_chunks):
    def gather_desc(k, b):
        return pltpu.make_async_copy(table_ref.at[src_slab.at[k]],
                                     bufs.at[b], gsems[b])

    gather_desc(0, 0).start()

    def body(j, carry):
        for b in range(2):
            k = j * 2 + b
            nb = 1 - b

            @pl.when(k + 1 < n_chunks)
            def _():
                gather_desc(k + 1, nb).start()

            gather_desc(k, b).wait()
            pltpu.sync_copy(bufs.at[b], acc_sp.at[dst_slab.at[k]], add=True)
        return carry

    lax.fori_loop(0, n_chunks // 2, body, 0)


def _sc_edges_wide(table_hbm, src2d_hbm, dst2d_hbm, out_hbm,
                   acc_sp, src_slab, dst_slab, bufs, gs0, gs1):
    c = lax.axis_index("c")
    s = lax.axis_index("s")
    np_rows = acc_sp.shape[0]
    rows_per = np_rows // NS
    n_chunks = src2d_hbm.shape[0] // NS
    r0 = s * rows_per
    pltpu.sync_copy(src2d_hbm.at[pl.ds(s * n_chunks, n_chunks)], src_slab)
    pltpu.sync_copy(dst2d_hbm.at[pl.ds(s * n_chunks, n_chunks)], dst_slab)
    pltpu.sync_copy(table_hbm.at[c].at[pl.ds(r0, rows_per)],
                    acc_sp.at[pl.ds(r0, rows_per)])
    plsc.subcore_barrier()
    _edge_pipeline(table_hbm.at[c], acc_sp, src_slab, dst_slab, bufs,
                   [gs0, gs1], n_chunks)
    plsc.subcore_barrier()
    pltpu.sync_copy(acc_sp.at[pl.ds(r0, rows_per)],
                    out_hbm.at[c, pl.ds(r0, rows_per)])


def _sc_edges_narrow(z_hbm, src2d_hbm, dst2d_hbm, out_hbm,
                     acc_sp, src_slab, dst_slab, bufs, gs0, gs1):
    c = lax.axis_index("c")
    s = lax.axis_index("s")
    np_rows = acc_sp.shape[0]
    rows_per = np_rows // NS
    n_chunks = src2d_hbm.shape[0] // (NC * NS)
    r0 = s * rows_per
    w = c * NS + s
    pltpu.sync_copy(src2d_hbm.at[pl.ds(w * n_chunks, n_chunks)], src_slab)
    pltpu.sync_copy(dst2d_hbm.at[pl.ds(w * n_chunks, n_chunks)], dst_slab)
    pltpu.sync_copy(z_hbm.at[pl.ds(r0, rows_per)],
                    acc_sp.at[pl.ds(r0, rows_per)])
    plsc.subcore_barrier()
    _edge_pipeline(z_hbm, acc_sp, src_slab, dst_slab, bufs,
                   [gs0, gs1], n_chunks)
    plsc.subcore_barrier()
    pltpu.sync_copy(acc_sp.at[pl.ds(r0, rows_per)],
                    out_hbm.at[c, pl.ds(r0, rows_per)])


def _tc_scale_in(x_ref, w1_ref, deg_ref, t_ref, dis_ref):
    deg = deg_ref[0, :, 0:1] + deg_ref[1, :, 0:1]
    dis = lax.rsqrt(deg)
    h = jnp.dot(x_ref[...], w1_ref[...], precision=lax.Precision.HIGHEST,
                preferred_element_type=jnp.float32)
    hp = h * dis
    t_ref[0] = hp[:, :64].astype(t_ref.dtype)
    t_ref[1] = hp[:, 64:].astype(t_ref.dtype)
    dis_ref[...] = jnp.broadcast_to(dis, dis_ref.shape)


def _tc_mid(agg_ref, dis_ref, b1_ref, w2_ref, z_ref):
    dis = dis_ref[:, 0:1]
    agg = jnp.concatenate([agg_ref[0], agg_ref[1]], axis=1).astype(jnp.float32)
    out1 = jnp.maximum(agg * dis + b1_ref[...], 0.0)
    z_ref[...] = jnp.dot(out1 * dis, w2_ref[...],
                         precision=lax.Precision.HIGHEST,
                         preferred_element_type=jnp.float32)


def _tc_combine_pool(p_ref, z_ref, dis_ref, b2_ref, batch_ref,
                     out_ref, *, n, g):
    dis = dis_ref[:, 0:1]
    agg = p_ref[0] + p_ref[1] - z_ref[...]
    h2 = agg * dis + b2_ref[...]
    h2 = h2[:n, :]
    gids = lax.broadcasted_iota(jnp.int32, (g, n), 0)
    onehot = (gids == batch_ref[...]).astype(jnp.float32)
    out_ref[...] = jnp.dot(onehot, h2, precision=lax.Precision.HIGHEST,
                           preferred_element_type=jnp.float32)


def kernel(x, edge_index, batch, W1, b1, W2, b2):
    n, d_in = x.shape
    d_hid = W1.shape[1]
    d_out = W2.shape[1]
    g = 64
    e = edge_index.shape[1]
    f32 = jnp.float32

    n_pad = _ceil_to(n, NS * CHUNK)
    e_pad = _ceil_to(e, 2 * NC * NS * CHUNK)
    dh = d_hid // NC

    x_p = jnp.pad(x, ((0, n_pad - n), (0, 0)))
    src2d = jnp.pad(edge_index[0].astype(jnp.int32),
                    (0, e_pad - e)).reshape(e_pad // CHUNK, CHUNK)
    dst2d = jnp.pad(edge_index[1].astype(jnp.int32), (0, e_pad - e),
                    constant_values=n).reshape(e_pad // CHUNK, CHUNK)
    deg_init = jnp.stack([jnp.ones((n_pad, 16), f32),
                          jnp.zeros((n_pad, 16), f32)])
    ones_rows = jnp.ones((CHUNK, 16), f32)
    b1_r = b1.reshape(1, d_hid)
    b2_r = b2.reshape(1, d_out)
    batch_r = batch.astype(jnp.int32).reshape(1, n)

    n_chunks3 = e_pad // CHUNK // NS
    n_chunks5 = e_pad // CHUNK // (NC * NS)

    sc_params = pltpu.CompilerParams(use_tc_tiling_on_sc=False)

    deg_parts = pl.kernel(
        _sc_degree,
        out_type=jax.ShapeDtypeStruct((NC, n_pad, 16), f32),
        mesh=_mesh(),
        compiler_params=sc_params,
        scratch_types=[
            pltpu.VMEM_SHARED((n_pad, 16), f32),
            pltpu.VMEM((n_chunks5, CHUNK), jnp.int32),
            pltpu.VMEM((CHUNK, 16), f32),
            pltpu.SemaphoreType.DMA,
        ],
    )(dst2d, deg_init, ones_rows)

    rows_blk = n_pad // 8
    table, dis8 = pl.pallas_call(
        _tc_scale_in,
        grid=(8,),
        in_specs=[
            pl.BlockSpec((rows_blk, d_in), lambda i: (i, 0)),
            pl.BlockSpec((d_in, d_hid), lambda i: (0, 0)),
            pl.BlockSpec((NC, rows_blk, 16), lambda i: (0, i, 0)),
        ],
        out_specs=[
            pl.BlockSpec((NC, rows_blk, dh), lambda i: (0, i, 0)),
            pl.BlockSpec((rows_blk, 8), lambda i: (i, 0)),
        ],
        out_shape=[
            jax.ShapeDtypeStruct((NC, n_pad, dh), jnp.bfloat16),
            jax.ShapeDtypeStruct((n_pad, 8), f32),
        ],
    )(x_p, W1, deg_parts)

    agg1 = pl.kernel(
        _sc_edges_wide,
        out_type=jax.ShapeDtypeStruct((NC, n_pad, dh), jnp.bfloat16),
        mesh=_mesh(),
        compiler_params=sc_params,
        scratch_types=[
            pltpu.VMEM_SHARED((n_pad, dh), jnp.bfloat16),
            pltpu.VMEM((n_chunks3, CHUNK), jnp.int32),
            pltpu.VMEM((n_chunks3, CHUNK), jnp.int32),
            pltpu.VMEM((2, CHUNK, dh), jnp.bfloat16),
            pltpu.SemaphoreType.DMA,
            pltpu.SemaphoreType.DMA,
        ],
    )(table, src2d, dst2d)

    z = pl.pallas_call(
        _tc_mid,
        grid=(8,),
        in_specs=[
            pl.BlockSpec((NC, rows_blk, dh), lambda i: (0, i, 0)),
            pl.BlockSpec((rows_blk, 8), lambda i: (i, 0)),
            pl.BlockSpec((1, d_hid), lambda i: (0, 0)),
            pl.BlockSpec((d_hid, d_out), lambda i: (0, 0)),
        ],
        out_specs=pl.BlockSpec((rows_blk, d_out), lambda i: (i, 0)),
        out_shape=jax.ShapeDtypeStruct((n_pad, d_out), f32),
    )(agg1, dis8, b1_r, W2)

    agg2 = pl.kernel(
        _sc_edges_narrow,
        out_type=jax.ShapeDtypeStruct((NC, n_pad, d_out), f32),
        mesh=_mesh(),
        compiler_params=sc_params,
        scratch_types=[
            pltpu.VMEM_SHARED((n_pad, d_out), f32),
            pltpu.VMEM((n_chunks5, CHUNK), jnp.int32),
            pltpu.VMEM((n_chunks5, CHUNK), jnp.int32),
            pltpu.VMEM((2, CHUNK, d_out), f32),
            pltpu.SemaphoreType.DMA,
            pltpu.SemaphoreType.DMA,
        ],
    )(z, src2d, dst2d)

    out = pl.pallas_call(
        functools.partial(_tc_combine_pool, n=n, g=g),
        in_specs=[
            pl.BlockSpec((NC, n_pad, d_out), lambda: (0, 0, 0)),
            pl.BlockSpec((n_pad, d_out), lambda: (0, 0)),
            pl.BlockSpec((n_pad, 8), lambda: (0, 0)),
            pl.BlockSpec((1, d_out), lambda: (0, 0)),
            pl.BlockSpec((1, n), lambda: (0, 0)),
        ],
        out_specs=pl.BlockSpec((g, d_out), lambda: (0, 0)),
        out_shape=jax.ShapeDtypeStruct((g, d_out), f32),
    )(agg2, z, dis8, b2_r, batch_r)

    return out


# speedup vs baseline: 1.3544x; 1.0263x over previous
import functools

import jax
import jax.numpy as jnp
from jax import lax
from jax.experimental import pallas as pl
from jax.experimental.pallas import tpu as pltpu
from jax.experimental.pallas import tpu_sc as plsc

NC = 2
NS = 16
CHUNK = 128


@functools.cache
def _mesh():
    return plsc.VectorSubcoreMesh(core_axis_name="c", subcore_axis_name="s",
                                  num_cores=NC, num_subcores=NS)


def _ceil_to(x, m):
    return (x + m - 1) // m * m


def _sc_degree(ei3d_hbm, init_hbm, ones_hbm, out_hbm,
               acc_sp, dst_slab, ones_v, sem):
    c = lax.axis_index("c")
    s = lax.axis_index("s")
    np_rows = acc_sp.shape[0]
    rows_per = np_rows // NS
    n_chunks = ei3d_hbm.shape[1] // (NC * NS)
    r0 = s * rows_per
    w = c * NS + s
    pltpu.sync_copy(ei3d_hbm.at[1].at[pl.ds(w * n_chunks, n_chunks)], dst_slab)
    pltpu.sync_copy(init_hbm.at[c, pl.ds(r0, rows_per)],
                    acc_sp.at[pl.ds(r0, rows_per)])
    pltpu.sync_copy(ones_hbm, ones_v)
    plsc.subcore_barrier()

    grp = 8

    def body(j, carry):
        for b in range(grp):
            pltpu.async_copy(ones_v, acc_sp.at[dst_slab.at[j * grp + b]],
                             sem, add=True)
        for b in range(grp):
            pltpu.make_async_copy(ones_v, acc_sp.at[dst_slab.at[0]],
                                  sem).wait()
        return carry

    lax.fori_loop(0, n_chunks // grp, body, 0)
    plsc.subcore_barrier()
    pltpu.sync_copy(acc_sp.at[pl.ds(r0, rows_per)],
                    out_hbm.at[c, pl.ds(r0, rows_per)])


def _edge_pipeline(table_ref, acc_sp, src_slab, dst_slab, bufs,
                   gsems, n_chunks):
    def gather_desc(k, b):
        return pltpu.make_async_copy(table_ref.at[src_slab.at[k]],
                                     bufs.at[b], gsems[b])

    gather_desc(0, 0).start()

    def body(j, carry):
        for b in range(2):
            k = j * 2 + b
            nb = 1 - b

            @pl.when(k + 1 < n_chunks)
            def _():
                gather_desc(k + 1, nb).start()

            gather_desc(k, b).wait()
            pltpu.sync_copy(bufs.at[b], acc_sp.at[dst_slab.at[k]], add=True)
        return carry

    lax.fori_loop(0, n_chunks // 2, body, 0)


def _sc_edges_wide(table_hbm, ei3d_hbm, out_hbm,
                   acc_sp, src_slab, dst_slab, bufs, gs0, gs1):
    c = lax.axis_index("c")
    s = lax.axis_index("s")
    np_rows = acc_sp.shape[0]
    rows_per = np_rows // NS
    n_chunks = ei3d_hbm.shape[1] // NS
    r0 = s * rows_per
    pltpu.sync_copy(ei3d_hbm.at[0].at[pl.ds(s * n_chunks, n_chunks)], src_slab)
    pltpu.sync_copy(ei3d_hbm.at[1].at[pl.ds(s * n_chunks, n_chunks)], dst_slab)
    pltpu.sync_copy(table_hbm.at[c].at[pl.ds(r0, rows_per)],
                    acc_sp.at[pl.ds(r0, rows_per)])
    plsc.subcore_barrier()
    _edge_pipeline(table_hbm.at[c], acc_sp, src_slab, dst_slab, bufs,
                   [gs0, gs1], n_chunks)
    plsc.subcore_barrier()
    pltpu.sync_copy(acc_sp.at[pl.ds(r0, rows_per)],
                    out_hbm.at[c, pl.ds(r0, rows_per)])


def _sc_edges_narrow(z_hbm, ei3d_hbm, out_hbm,
                     acc_sp, src_slab, dst_slab, bufs, gs0, gs1):
    c = lax.axis_index("c")
    s = lax.axis_index("s")
    np_rows = acc_sp.shape[0]
    rows_per = np_rows // NS
    n_chunks = ei3d_hbm.shape[1] // (NC * NS)
    r0 = s * rows_per
    w = c * NS + s
    pltpu.sync_copy(ei3d_hbm.at[0].at[pl.ds(w * n_chunks, n_chunks)], src_slab)
    pltpu.sync_copy(ei3d_hbm.at[1].at[pl.ds(w * n_chunks, n_chunks)], dst_slab)
    pltpu.sync_copy(z_hbm.at[pl.ds(r0, rows_per)],
                    acc_sp.at[pl.ds(r0, rows_per)])
    plsc.subcore_barrier()
    _edge_pipeline(z_hbm, acc_sp, src_slab, dst_slab, bufs,
                   [gs0, gs1], n_chunks)
    plsc.subcore_barrier()
    pltpu.sync_copy(acc_sp.at[pl.ds(r0, rows_per)],
                    out_hbm.at[c, pl.ds(r0, rows_per)])


def _tc_scale_in(x_ref, w1_ref, deg_ref, t_ref, dis_ref):
    deg = deg_ref[0, :, 0:1] + deg_ref[1, :, 0:1]
    dis = lax.rsqrt(deg)
    h = jnp.dot(x_ref[...], w1_ref[...], precision=lax.Precision.HIGHEST,
                preferred_element_type=jnp.float32)
    hp = h * dis
    t_ref[0] = hp[:, :64].astype(t_ref.dtype)
    t_ref[1] = hp[:, 64:].astype(t_ref.dtype)
    dis_ref[...] = jnp.broadcast_to(dis, dis_ref.shape)


def _tc_mid(agg_ref, dis_ref, b1_ref, w2_ref, z_ref):
    dis = dis_ref[:, 0:1]
    agg = jnp.concatenate([agg_ref[0], agg_ref[1]], axis=1).astype(jnp.float32)
    out1 = jnp.maximum(agg * dis + b1_ref[...], 0.0)
    z_ref[...] = jnp.dot(out1 * dis, w2_ref[...],
                         precision=lax.Precision.HIGHEST,
                         preferred_element_type=jnp.float32)


def _tc_combine_pool(p_ref, z_ref, dis_ref, b2_ref, batch_ref,
                     out_ref, *, n, g):
    dis = dis_ref[:, 0:1]
    agg = p_ref[0] + p_ref[1] - z_ref[...]
    h2 = agg * dis + b2_ref[...]
    h2 = h2[:n, :]
    gids = lax.broadcasted_iota(jnp.int32, (g, n), 0)
    onehot = (gids == batch_ref[...]).astype(jnp.float32)
    out_ref[...] = jnp.dot(onehot, h2, precision=lax.Precision.HIGHEST,
                           preferred_element_type=jnp.float32)


def kernel(x, edge_index, batch, W1, b1, W2, b2):
    n, d_in = x.shape
    d_hid = W1.shape[1]
    d_out = W2.shape[1]
    g = 64
    e = edge_index.shape[1]
    f32 = jnp.float32

    n_pad = _ceil_to(n, NS * CHUNK)
    e_pad = _ceil_to(e, 2 * NC * NS * CHUNK)
    dh = d_hid // NC

    x_p = jnp.pad(x, ((0, n_pad - n), (0, 0)))
    ei3d = jnp.pad(edge_index.astype(jnp.int32), ((0, 0), (0, e_pad - e)),
                   constant_values=n).reshape(2, e_pad // CHUNK, CHUNK)
    deg_init = jnp.stack([jnp.ones((n_pad, 16), f32),
                          jnp.zeros((n_pad, 16), f32)])
    ones_rows = jnp.ones((CHUNK, 16), f32)
    b1_r = b1.reshape(1, d_hid)
    b2_r = b2.reshape(1, d_out)
    batch_r = batch.astype(jnp.int32).reshape(1, n)

    n_chunks3 = e_pad // CHUNK // NS
    n_chunks5 = e_pad // CHUNK // (NC * NS)

    sc_params = pltpu.CompilerParams(use_tc_tiling_on_sc=False)

    deg_parts = pl.kernel(
        _sc_degree,
        out_type=jax.ShapeDtypeStruct((NC, n_pad, 16), f32),
        mesh=_mesh(),
        compiler_params=sc_params,
        scratch_types=[
            pltpu.VMEM_SHARED((n_pad, 16), f32),
            pltpu.VMEM((n_chunks5, CHUNK), jnp.int32),
            pltpu.VMEM((CHUNK, 16), f32),
            pltpu.SemaphoreType.DMA,
        ],
    )(ei3d, deg_init, ones_rows)

    rows_blk = n_pad // 8
    table, dis8 = pl.pallas_call(
        _tc_scale_in,
        grid=(8,),
        in_specs=[
            pl.BlockSpec((rows_blk, d_in), lambda i: (i, 0)),
            pl.BlockSpec((d_in, d_hid), lambda i: (0, 0)),
            pl.BlockSpec((NC, rows_blk, 16), lambda i: (0, i, 0)),
        ],
        out_specs=[
            pl.BlockSpec((NC, rows_blk, dh), lambda i: (0, i, 0)),
            pl.BlockSpec((rows_blk, 8), lambda i: (i, 0)),
        ],
        out_shape=[
            jax.ShapeDtypeStruct((NC, n_pad, dh), jnp.bfloat16),
            jax.ShapeDtypeStruct((n_pad, 8), f32),
        ],
    )(x_p, W1, deg_parts)

    agg1 = pl.kernel(
        _sc_edges_wide,
        out_type=jax.ShapeDtypeStruct((NC, n_pad, dh), jnp.bfloat16),
        mesh=_mesh(),
        compiler_params=sc_params,
        scratch_types=[
            pltpu.VMEM_SHARED((n_pad, dh), jnp.bfloat16),
            pltpu.VMEM((n_chunks3, CHUNK), jnp.int32),
            pltpu.VMEM((n_chunks3, CHUNK), jnp.int32),
            pltpu.VMEM((2, CHUNK, dh), jnp.bfloat16),
            pltpu.SemaphoreType.DMA,
            pltpu.SemaphoreType.DMA,
        ],
    )(table, ei3d)

    z = pl.pallas_call(
        _tc_mid,
        grid=(8,),
        in_specs=[
            pl.BlockSpec((NC, rows_blk, dh), lambda i: (0, i, 0)),
            pl.BlockSpec((rows_blk, 8), lambda i: (i, 0)),
            pl.BlockSpec((1, d_hid), lambda i: (0, 0)),
            pl.BlockSpec((d_hid, d_out), lambda i: (0, 0)),
        ],
        out_specs=pl.BlockSpec((rows_blk, d_out), lambda i: (i, 0)),
        out_shape=jax.ShapeDtypeStruct((n_pad, d_out), f32),
    )(agg1, dis8, b1_r, W2)

    agg2 = pl.kernel(
        _sc_edges_narrow,
        out_type=jax.ShapeDtypeStruct((NC, n_pad, d_out), f32),
        mesh=_mesh(),
        compiler_params=sc_params,
        scratch_types=[
            pltpu.VMEM_SHARED((n_pad, d_out), f32),
            pltpu.VMEM((n_chunks5, CHUNK), jnp.int32),
            pltpu.VMEM((n_chunks5, CHUNK), jnp.int32),
            pltpu.VMEM((2, CHUNK, d_out), f32),
            pltpu.SemaphoreType.DMA,
            pltpu.SemaphoreType.DMA,
        ],
    )(z, ei3d)

    out = pl.pallas_call(
        functools.partial(_tc_combine_pool, n=n, g=g),
        in_specs=[
            pl.BlockSpec((NC, n_pad, d_out), lambda: (0, 0, 0)),
            pl.BlockSpec((n_pad, d_out), lambda: (0, 0)),
            pl.BlockSpec((n_pad, 8), lambda: (0, 0)),
            pl.BlockSpec((1, d_out), lambda: (0, 0)),
            pl.BlockSpec((1, n), lambda: (0, 0)),
        ],
        out_specs=pl.BlockSpec((g, d_out), lambda: (0, 0)),
        out_shape=jax.ShapeDtypeStruct((g, d_out), f32),
    )(agg2, z, dis8, b2_r, batch_r)

    return out
